# baseline (speedup 1.0000x reference)
import jax
import jax.numpy as jnp
from jax import lax
from jax.experimental import pallas as pl
from jax.experimental.pallas import tpu as pltpu

_K = 8


def _router_body(x_ref, w_ref, out_w_ref, out_i_ref):
    xb = x_ref[...]
    wb = w_ref[...]
    logits = jnp.dot(xb, wb, preferred_element_type=jnp.float32)
    bt, E = logits.shape
    m = jnp.max(logits, axis=1, keepdims=True)
    e = jnp.exp(logits - m)
    z = jnp.sum(e, axis=1, keepdims=True)
    p = e / z

    iota_e = lax.broadcasted_iota(jnp.int32, (bt, E), 1)
    vals = []
    idxs = []
    cur = p
    for _ in range(_K):
        mk = jnp.max(cur, axis=1, keepdims=True)
        hit = cur == mk
        ik = jnp.min(jnp.where(hit, iota_e, E), axis=1, keepdims=True)
        vals.append(mk)
        idxs.append(ik)
        cur = jnp.where(iota_e == ik, -jnp.inf, cur)

    v = jnp.concatenate(vals, axis=1)
    i = jnp.concatenate(idxs, axis=1)
    e2 = jnp.exp(v - v[:, :1])
    w = e2 / jnp.sum(e2, axis=1, keepdims=True)
    out_w_ref[...] = w
    out_i_ref[...] = i


def kernel(x, kernel_DE):
    B, T, D = x.shape
    E = kernel_DE.shape[1]
    BT = B * T
    bt = 512
    x2 = x.reshape(BT, D)

    w_out, i_out = pl.pallas_call(
        _router_body,
        grid=(BT // bt,),
        in_specs=[
            pl.BlockSpec((bt, D), lambda i: (i, 0)),
            pl.BlockSpec((D, E), lambda i: (0, 0)),
        ],
        out_specs=[
            pl.BlockSpec((bt, _K), lambda i: (i, 0)),
            pl.BlockSpec((bt, _K), lambda i: (i, 0)),
        ],
        out_shape=[
            jax.ShapeDtypeStruct((BT, _K), jnp.float32),
            jax.ShapeDtypeStruct((BT, _K), jnp.int32),
        ],
    )(x2, kernel_DE)

    return w_out.reshape(B, T, _K), i_out.reshape(B, T, _K)


# speedup vs baseline: 2.7640x; 2.7640x over previous
import jax
import jax.numpy as jnp
from jax import lax
from jax.experimental import pallas as pl
from jax.experimental.pallas import tpu as pltpu

_K = 8


def _router_body(x_ref, w_ref, out_w_ref, out_i_ref):
    xb = x_ref[...]
    wb = w_ref[...]
    logits = jnp.dot(xb, wb, preferred_element_type=jnp.float32)
    bt, E = logits.shape
    lt = logits.T
    m = jnp.max(lt, axis=0, keepdims=True)
    e = jnp.exp(lt - m)
    z = jnp.sum(e, axis=0, keepdims=True)
    p = e / z

    iota_e = lax.broadcasted_iota(jnp.int32, (E, bt), 0)
    vals = []
    idxs = []
    cur = p
    for _ in range(_K):
        mk = jnp.max(cur, axis=0, keepdims=True)
        hit = cur == mk
        ik = jnp.min(jnp.where(hit, iota_e, E), axis=0, keepdims=True)
        vals.append(mk)
        idxs.append(ik)
        cur = jnp.where(iota_e == ik, -jnp.inf, cur)

    v = jnp.concatenate(vals, axis=0)
    i = jnp.concatenate(idxs, axis=0)
    e2 = jnp.exp(v - v[:1])
    w = e2 / jnp.sum(e2, axis=0, keepdims=True)
    out_w_ref[...] = w
    out_i_ref[...] = i


def kernel(x, kernel_DE):
    B, T, D = x.shape
    E = kernel_DE.shape[1]
    BT = B * T
    bt = 1024
    x2 = x.reshape(BT, D)

    w_out, i_out = pl.pallas_call(
        _router_body,
        grid=(BT // bt,),
        in_specs=[
            pl.BlockSpec((bt, D), lambda i: (i, 0)),
            pl.BlockSpec((D, E), lambda i: (0, 0)),
        ],
        out_specs=[
            pl.BlockSpec((_K, bt), lambda i: (0, i)),
            pl.BlockSpec((_K, bt), lambda i: (0, i)),
        ],
        out_shape=[
            jax.ShapeDtypeStruct((_K, BT), jnp.float32),
            jax.ShapeDtypeStruct((_K, BT), jnp.int32),
        ],
    )(x2, kernel_DE)

    return w_out.T.reshape(B, T, _K), i_out.T.reshape(B, T, _K)


# speedup vs baseline: 2.8626x; 1.0357x over previous
import jax
import jax.numpy as jnp
from jax import lax
from jax.experimental import pallas as pl
from jax.experimental.pallas import tpu as pltpu

_K = 8


def _router_body(x_ref, w_ref, out_w_ref, out_i_ref):
    xb = x_ref[...]
    wb = w_ref[...]
    logits = jnp.dot(xb, wb, preferred_element_type=jnp.float32)
    bt, E = logits.shape
    lt = logits.T
    m = jnp.max(lt, axis=0, keepdims=True)
    e = jnp.exp(lt - m)
    z = jnp.sum(e, axis=0, keepdims=True)
    p = e / z

    iota_e = lax.broadcasted_iota(jnp.int32, (E, bt), 0)
    vals = []
    idxs = []
    cur = p
    for _ in range(_K):
        mk = jnp.max(cur, axis=0, keepdims=True)
        hit = cur == mk
        ik = jnp.min(jnp.where(hit, iota_e, E), axis=0, keepdims=True)
        vals.append(mk)
        idxs.append(ik)
        cur = jnp.where(iota_e == ik, -jnp.inf, cur)

    v = jnp.concatenate(vals, axis=0)
    i = jnp.concatenate(idxs, axis=0)
    e2 = jnp.exp(v - v[:1])
    w = e2 / jnp.sum(e2, axis=0, keepdims=True)
    out_w_ref[...] = w
    out_i_ref[...] = i


def kernel(x, kernel_DE):
    B, T, D = x.shape
    E = kernel_DE.shape[1]
    BT = B * T
    bt = 2048
    x2 = x.reshape(BT, D)

    w_out, i_out = pl.pallas_call(
        _router_body,
        grid=(BT // bt,),
        in_specs=[
            pl.BlockSpec((bt, D), lambda i: (i, 0)),
            pl.BlockSpec((D, E), lambda i: (0, 0)),
        ],
        out_specs=[
            pl.BlockSpec((_K, bt), lambda i: (0, i)),
            pl.BlockSpec((_K, bt), lambda i: (0, i)),
        ],
        out_shape=[
            jax.ShapeDtypeStruct((_K, BT), jnp.float32),
            jax.ShapeDtypeStruct((_K, BT), jnp.int32),
        ],
    )(x2, kernel_DE)

    return w_out.T.reshape(B, T, _K), i_out.T.reshape(B, T, _K)
